# baseline (speedup 1.0000x reference)
import functools

import jax
import jax.numpy as jnp
from jax import lax
from jax.experimental import pallas as pl
from jax.experimental.pallas import tpu as pltpu
from jax.experimental.pallas import tpu_sc as plsc

B, P, C, K, R = 5, 2000, 128, 16, 4
N = B * P
ROWS = 400
RB = P // ROWS

_SC = plsc.get_sparse_core_info()
_NC, _NS = _SC.num_cores, _SC.num_subcores
NW = _NC * _NS
PC = 2048
PPW = PC // NW
CH = 8
NCHUNK = PPW // CH


def _dot(a, b):
    return jax.lax.dot_general(a, b, (((1,), (0,)), ((), ())),
                               preferred_element_type=jnp.float32)


def _sq_body(x_ref, sq_ref):
    x = x_ref[...]
    sq_ref[...] = jnp.sum(x * x, axis=1, keepdims=True)


def _rowsq(x):
    return pl.pallas_call(
        _sq_body,
        grid=(N // ROWS,),
        in_specs=[pl.BlockSpec((ROWS, C), lambda r: (r, 0))],
        out_specs=pl.BlockSpec((ROWS, 1), lambda r: (r, 0)),
        out_shape=jax.ShapeDtypeStruct((N, 1), jnp.float32),
    )(x)


def _make_knn_body(T, want2):
    def body(xr_ref, xf_ref, sq_ref, *o_refs):
        xr = xr_ref[...]
        xf = xf_ref[...]
        sq = sq_ref[...]
        sqr = jnp.sum(xr * xr, axis=1, keepdims=True)
        d0 = sqr + sq - 2.0 * jax.lax.dot_general(
            xr, xf, (((1,), (1,)), ((), ())),
            preferred_element_type=jnp.float32)
        lane = jax.lax.broadcasted_iota(jnp.int32, (ROWS, P), 1)
        col16 = jax.lax.broadcasted_iota(jnp.int32, (ROWS, 16), 1)

        def step(t, carry):
            d, a1, a2 = carry
            m = jnp.min(d, axis=1, keepdims=True)
            sel = jnp.where(d == m, lane, jnp.int32(P))
            argi = jnp.min(sel, axis=1, keepdims=True)
            d = jnp.where(lane == argi, jnp.inf, d)
            a1 = jnp.where(col16 == (t - 1), argi, a1)
            if want2:
                upd2 = jnp.logical_and((t % 2) == 1, col16 == ((t - 1) // 2))
                a2 = jnp.where(upd2, argi, a2)
            return d, a1, a2

        z = jnp.zeros((ROWS, 16), jnp.int32)
        _, a1, a2 = lax.fori_loop(0, T, step, (d0, z, z))
        o_refs[0][...] = a1
        if want2:
            o_refs[1][...] = a2
    return body


@functools.cache
def _knn_cloud_fn(T, want2):
    n_out = 2 if want2 else 1
    return pl.pallas_call(
        _make_knn_body(T, want2),
        grid=(RB,),
        in_specs=[
            pl.BlockSpec((ROWS, C), lambda r: (r, 0)),
            pl.BlockSpec((P, C), lambda r: (0, 0)),
            pl.BlockSpec((1, P), lambda r: (0, 0)),
        ],
        out_specs=[pl.BlockSpec((ROWS, 16), lambda r: (r, 0))
                   for _ in range(n_out)],
        out_shape=[jax.ShapeDtypeStruct((P, 16), jnp.int32)
                   for _ in range(n_out)],
    )


IR = PPW * K // (CH * K)
NBUF = 4


def _gather_streams(tab_hbm, work, bufs, sems):
    n = len(work)
    cps = [None] * n

    def fire(j):
        ia, c, _, _ = work[j]
        cps[j] = pltpu.async_copy(tab_hbm.at[ia.at[c]], bufs[j % NBUF],
                                  sems[j % NBUF])

    for j in range(min(NBUF, n)):
        fire(j)
    for j in range(n):
        cps[j].wait()
        _, _, outr, off = work[j]
        pltpu.sync_copy(bufs[j % NBUF], outr.at[pl.ds(off, CH * K)])
        if j + NBUF < n:
            fire(j + NBUF)


def _sc_mesh():
    return plsc.VectorSubcoreMesh(core_axis_name="c", subcore_axis_name="s")


_G_SCRATCH = ([pltpu.VMEM((IR, CH * K), jnp.int32),
               pltpu.VMEM((IR, CH * K), jnp.int32)]
              + [pltpu.VMEM((CH * K, C), jnp.float32) for _ in range(NBUF)]
              + [pltpu.SemaphoreType.DMA for _ in range(NBUF)])


@functools.partial(
    pl.kernel, mesh=_sc_mesh(),
    out_type=[jax.ShapeDtypeStruct((PC * K, C), jnp.float32),
              jax.ShapeDtypeStruct((PC * K, C), jnp.float32)],
    scratch_types=_G_SCRATCH,
)
def _gather2_sc(idx1_hbm, idx2_hbm, tab_hbm, out1_hbm, out2_hbm,
                ia1, ia2, *bufs_sems):
    bufs, sems = bufs_sems[:NBUF], bufs_sems[NBUF:]
    wid = lax.axis_index("s") * _NC + lax.axis_index("c")
    rb = wid * PPW * K
    pltpu.sync_copy(idx1_hbm.at[pl.ds(wid * IR, IR)], ia1)
    pltpu.sync_copy(idx2_hbm.at[pl.ds(wid * IR, IR)], ia2)
    work = ([(ia1, c, out1_hbm, rb + c * CH * K) for c in range(IR)]
            + [(ia2, c, out2_hbm, rb + c * CH * K) for c in range(IR)])
    _gather_streams(tab_hbm, work, bufs, sems)


@functools.partial(
    pl.kernel, mesh=_sc_mesh(),
    out_type=jax.ShapeDtypeStruct((PC * K, C), jnp.float32),
    scratch_types=_G_SCRATCH,
)
def _gather1_sc(idx_hbm, tab_hbm, out_hbm, ia1, ia2, *bufs_sems):
    bufs, sems = bufs_sems[:NBUF], bufs_sems[NBUF:]
    wid = lax.axis_index("s") * _NC + lax.axis_index("c")
    rb = wid * PPW * K
    pltpu.sync_copy(idx_hbm.at[pl.ds(wid * IR, IR)], ia1)
    work = [(ia1, c, out_hbm, rb + c * CH * K) for c in range(IR)]
    _gather_streams(tab_hbm, work, bufs, sems)


def _pad_idx(idx):
    return jnp.pad(idx, ((0, PC - P), (0, 0))).reshape(-1, CH * K)


def _conv12_body(x_ref, xj1_ref, xj2_ref, we1_ref, be1_ref, we2_ref, be2_ref,
                 wf_ref, bf_ref, h_ref, sqh_ref):
    xi = x_ref[...]
    xi_rep = jnp.broadcast_to(xi[:, None, :], (ROWS, K, C)).reshape(ROWS * K, C)
    xi_b = xi_rep.astype(jnp.bfloat16)

    def conv(xj_ref, w_ref, b_ref):
        dj = (xj_ref[...] - xi_rep).astype(jnp.bfloat16)
        e = jnp.concatenate([xi_b, dj], axis=1)
        c = jax.lax.dot_general(e, w_ref[...], (((1,), (0,)), ((), ())),
                                preferred_element_type=jnp.float32)
        hk = jnp.maximum(c + b_ref[...], 0.0).reshape(ROWS, K, C)
        return jnp.max(hk, axis=1)

    h1 = conv(xj1_ref, we1_ref, be1_ref)
    h2 = conv(xj2_ref, we2_ref, be2_ref)
    s = h1 + h2
    h = jnp.maximum(_dot(s, wf_ref[...]) + bf_ref[...], 0.0) + xi
    h_ref[...] = h
    sqh_ref[...] = jnp.sum(h * h, axis=1, keepdims=True)


_conv12_cloud = pl.pallas_call(
    _conv12_body,
    grid=(RB,),
    in_specs=[
        pl.BlockSpec((ROWS, C), lambda r: (r, 0)),
        pl.BlockSpec((ROWS * K, C), lambda r: (r, 0)),
        pl.BlockSpec((ROWS * K, C), lambda r: (r, 0)),
        pl.BlockSpec((2 * C, C), lambda r: (0, 0)),
        pl.BlockSpec((1, C), lambda r: (0, 0)),
        pl.BlockSpec((2 * C, C), lambda r: (0, 0)),
        pl.BlockSpec((1, C), lambda r: (0, 0)),
        pl.BlockSpec((C, C), lambda r: (0, 0)),
        pl.BlockSpec((1, C), lambda r: (0, 0)),
    ],
    out_specs=[
        pl.BlockSpec((ROWS, C), lambda r: (r, 0)),
        pl.BlockSpec((ROWS, 1), lambda r: (r, 0)),
    ],
    out_shape=[
        jax.ShapeDtypeStruct((P, C), jnp.float32),
        jax.ShapeDtypeStruct((P, 1), jnp.float32),
    ],
)


def _tail_body(h_ref, xjs_ref, ws_ref, bs_ref, w1_ref, b1_ref, w2_ref, b2_ref,
               o_ref):
    hi = h_ref[...]
    hi_rep = jnp.broadcast_to(hi[:, None, :], (ROWS, K, C)).reshape(ROWS * K, C)
    dj = (xjs_ref[...] - hi_rep).astype(jnp.bfloat16)
    e = jnp.concatenate([hi_rep.astype(jnp.bfloat16), dj], axis=1)
    c = jax.lax.dot_general(e, ws_ref[...], (((1,), (0,)), ((), ())),
                            preferred_element_type=jnp.float32)
    hk = jnp.maximum(c + bs_ref[...], 0.0).reshape(ROWS, K, 4 * C)
    up = jnp.max(hk, axis=1).astype(jnp.bfloat16)
    for r in range(R):
        ur = up[:, C * r:C * (r + 1)]
        t = jnp.maximum(
            jax.lax.dot_general(ur, w1_ref[...], (((1,), (0,)), ((), ())),
                                preferred_element_type=jnp.float32)
            + b1_ref[...], 0.0).astype(jnp.bfloat16)
        o_ref[r] = jax.lax.dot_general(
            t, w2_ref[...], (((1,), (0,)), ((), ())),
            preferred_element_type=jnp.float32) + b2_ref[...]


_tail_cloud = pl.pallas_call(
    _tail_body,
    grid=(RB,),
    in_specs=[
        pl.BlockSpec((ROWS, C), lambda r: (r, 0)),
        pl.BlockSpec((ROWS * K, C), lambda r: (r, 0)),
        pl.BlockSpec((2 * C, 4 * C), lambda r: (0, 0)),
        pl.BlockSpec((1, 4 * C), lambda r: (0, 0)),
        pl.BlockSpec((C, C), lambda r: (0, 0)),
        pl.BlockSpec((1, C), lambda r: (0, 0)),
        pl.BlockSpec((C, 3), lambda r: (0, 0)),
        pl.BlockSpec((1, 3), lambda r: (0, 0)),
    ],
    out_specs=pl.BlockSpec((R, ROWS, 3), lambda r: (0, r, 0)),
    out_shape=jax.ShapeDtypeStruct((R, P, 3), jnp.float32),
)


def kernel(x, batch, We1, be1, We2, be2, Wf, bf, Ws, bs, Wr1, br1, Wr2, br2):
    be1r, be2r = be1.reshape(1, C), be2.reshape(1, C)
    bfr, bsr = bf.reshape(1, C), bs.reshape(1, 4 * C)
    br1r, br2r = br1.reshape(1, C), br2.reshape(1, 3)
    sqx = _rowsq(x).reshape(B, 1, P)
    knn2 = _knn_cloud_fn(32, True)
    knn1 = _knn_cloud_fn(17, False)
    outs = []
    for b in range(B):
        xb = x[b * P:(b + 1) * P]
        i1, i2 = knn2(xb, xb, sqx[b])
        xpad = jnp.pad(xb, ((0, PC - P), (0, 0)))
        XJ1, XJ2 = _gather2_sc(_pad_idx(i1), _pad_idx(i2), xpad)
        h, sqh = _conv12_cloud(xb, XJ1[:P * K], XJ2[:P * K],
                               We1, be1r, We2, be2r, Wf, bfr)
        (i_s,) = knn1(h, h, sqh.reshape(1, P))
        hpad = jnp.pad(h, ((0, PC - P), (0, 0)))
        XJS = _gather1_sc(_pad_idx(i_s), hpad)
        out4 = _tail_cloud(h, XJS[:P * K], Ws, bsr, Wr1, br1r, Wr2, br2r)
        outs.append(out4.transpose(1, 0, 2))
    return jnp.concatenate(outs, axis=0).reshape(N * R, 3)


# speedup vs baseline: 1.0122x; 1.0122x over previous
import functools

import jax
import jax.numpy as jnp
from jax import lax
from jax.experimental import pallas as pl
from jax.experimental.pallas import tpu as pltpu
from jax.experimental.pallas import tpu_sc as plsc

B, P, C, K, R = 5, 2000, 128, 16, 4
N = B * P
ROWS = 400
RB = P // ROWS

_SC = plsc.get_sparse_core_info()
_NC, _NS = _SC.num_cores, _SC.num_subcores
NW = _NC * _NS
PC = 2048
PPW = PC // NW
CH = 8
NCHUNK = PPW // CH


def _dot(a, b):
    return jax.lax.dot_general(a, b, (((1,), (0,)), ((), ())),
                               preferred_element_type=jnp.float32)


def _sq_body(x_ref, sq_ref):
    x = x_ref[...]
    sq_ref[...] = jnp.sum(x * x, axis=1, keepdims=True)


def _rowsq(x):
    return pl.pallas_call(
        _sq_body,
        grid=(N // ROWS,),
        in_specs=[pl.BlockSpec((ROWS, C), lambda r: (r, 0))],
        out_specs=pl.BlockSpec((ROWS, 1), lambda r: (r, 0)),
        out_shape=jax.ShapeDtypeStruct((N, 1), jnp.float32),
    )(x)


def _make_knn_body(T, want2):
    def body(xr_ref, xf_ref, sq_ref, *o_refs):
        xr = xr_ref[...]
        xf = xf_ref[...]
        sq = sq_ref[...]
        sqr = jnp.sum(xr * xr, axis=1, keepdims=True)
        d0 = sqr + sq - 2.0 * jax.lax.dot_general(
            xr, xf, (((1,), (1,)), ((), ())),
            preferred_element_type=jnp.float32)
        lane = jax.lax.broadcasted_iota(jnp.int32, (ROWS, P), 1)
        col16 = jax.lax.broadcasted_iota(jnp.int32, (ROWS, 16), 1)

        def step(t, carry):
            d, a1, a2 = carry
            m = jnp.min(d, axis=1, keepdims=True)
            sel = jnp.where(d == m, lane, jnp.int32(P))
            argi = jnp.min(sel, axis=1, keepdims=True)
            d = jnp.where(lane == argi, jnp.inf, d)
            a1 = jnp.where(col16 == (t - 1), argi, a1)
            if want2:
                upd2 = jnp.logical_and((t % 2) == 1, col16 == ((t - 1) // 2))
                a2 = jnp.where(upd2, argi, a2)
            return d, a1, a2

        z = jnp.zeros((ROWS, 16), jnp.int32)
        _, a1, a2 = lax.fori_loop(0, T, step, (d0, z, z))
        o_refs[0][...] = a1
        if want2:
            o_refs[1][...] = a2
    return body


@functools.cache
def _knn_cloud_fn(T, want2):
    n_out = 2 if want2 else 1
    return pl.pallas_call(
        _make_knn_body(T, want2),
        grid=(RB,),
        in_specs=[
            pl.BlockSpec((ROWS, C), lambda r: (r, 0)),
            pl.BlockSpec((P, C), lambda r: (0, 0)),
            pl.BlockSpec((1, P), lambda r: (0, 0)),
        ],
        out_specs=[pl.BlockSpec((ROWS, 16), lambda r: (r, 0))
                   for _ in range(n_out)],
        out_shape=[jax.ShapeDtypeStruct((P, 16), jnp.int32)
                   for _ in range(n_out)],
    )


def _gather_loop(idx_hbm, tab_hbm, out_hbm, idx_v, rows_v, sem, base):
    def chunk_body(i, carry):
        rbase = (base + i * CH) * K
        pltpu.sync_copy(idx_hbm.at[pl.ds(rbase, CH * K)], idx_v)
        pltpu.async_copy(tab_hbm.at[idx_v], rows_v, sem).wait()
        pltpu.sync_copy(rows_v, out_hbm.at[pl.ds(rbase, CH * K)])
        return carry

    lax.fori_loop(0, NCHUNK, chunk_body, 0)


def _sc_mesh():
    return plsc.VectorSubcoreMesh(core_axis_name="c", subcore_axis_name="s")


@functools.partial(
    pl.kernel, mesh=_sc_mesh(),
    out_type=[jax.ShapeDtypeStruct((PC * K, C), jnp.float32),
              jax.ShapeDtypeStruct((PC * K, C), jnp.float32)],
    scratch_types=[
        pltpu.VMEM((CH * K,), jnp.int32),
        pltpu.VMEM((CH * K, C), jnp.float32),
        pltpu.SemaphoreType.DMA,
    ],
)
def _gather2_sc(idx1_hbm, idx2_hbm, tab_hbm, out1_hbm, out2_hbm,
                idx_v, rows_v, sem):
    wid = lax.axis_index("s") * _NC + lax.axis_index("c")
    base = wid * PPW
    _gather_loop(idx1_hbm, tab_hbm, out1_hbm, idx_v, rows_v, sem, base)
    _gather_loop(idx2_hbm, tab_hbm, out2_hbm, idx_v, rows_v, sem, base)


@functools.partial(
    pl.kernel, mesh=_sc_mesh(),
    out_type=jax.ShapeDtypeStruct((PC * K, C), jnp.float32),
    scratch_types=[
        pltpu.VMEM((CH * K,), jnp.int32),
        pltpu.VMEM((CH * K, C), jnp.float32),
        pltpu.SemaphoreType.DMA,
    ],
)
def _gather1_sc(idx_hbm, tab_hbm, out_hbm, idx_v, rows_v, sem):
    wid = lax.axis_index("s") * _NC + lax.axis_index("c")
    _gather_loop(idx_hbm, tab_hbm, out_hbm, idx_v, rows_v, sem, wid * PPW)


def _pad_idx(idx):
    return jnp.pad(idx, ((0, PC - P), (0, 0))).reshape(-1)


def _conv12_body(x_ref, xj1_ref, xj2_ref, we1_ref, be1_ref, we2_ref, be2_ref,
                 wf_ref, bf_ref, h_ref, sqh_ref):
    xi = x_ref[...]
    xi_rep = jnp.broadcast_to(xi[:, None, :], (ROWS, K, C)).reshape(ROWS * K, C)
    xi_b = xi_rep.astype(jnp.bfloat16)

    def conv(xj_ref, w_ref, b_ref):
        dj = (xj_ref[...] - xi_rep).astype(jnp.bfloat16)
        e = jnp.concatenate([xi_b, dj], axis=1)
        c = jax.lax.dot_general(e, w_ref[...], (((1,), (0,)), ((), ())),
                                preferred_element_type=jnp.float32)
        hk = jnp.maximum(c + b_ref[...], 0.0).reshape(ROWS, K, C)
        return jnp.max(hk, axis=1)

    h1 = conv(xj1_ref, we1_ref, be1_ref)
    h2 = conv(xj2_ref, we2_ref, be2_ref)
    s = h1 + h2
    h = jnp.maximum(_dot(s, wf_ref[...]) + bf_ref[...], 0.0) + xi
    h_ref[...] = h
    sqh_ref[...] = jnp.sum(h * h, axis=1, keepdims=True)


_conv12_cloud = pl.pallas_call(
    _conv12_body,
    grid=(RB,),
    in_specs=[
        pl.BlockSpec((ROWS, C), lambda r: (r, 0)),
        pl.BlockSpec((ROWS * K, C), lambda r: (r, 0)),
        pl.BlockSpec((ROWS * K, C), lambda r: (r, 0)),
        pl.BlockSpec((2 * C, C), lambda r: (0, 0)),
        pl.BlockSpec((1, C), lambda r: (0, 0)),
        pl.BlockSpec((2 * C, C), lambda r: (0, 0)),
        pl.BlockSpec((1, C), lambda r: (0, 0)),
        pl.BlockSpec((C, C), lambda r: (0, 0)),
        pl.BlockSpec((1, C), lambda r: (0, 0)),
    ],
    out_specs=[
        pl.BlockSpec((ROWS, C), lambda r: (r, 0)),
        pl.BlockSpec((ROWS, 1), lambda r: (r, 0)),
    ],
    out_shape=[
        jax.ShapeDtypeStruct((P, C), jnp.float32),
        jax.ShapeDtypeStruct((P, 1), jnp.float32),
    ],
)


def _tail_body(h_ref, xjs_ref, ws_ref, bs_ref, w1_ref, b1_ref, w2_ref, b2_ref,
               o_ref):
    hi = h_ref[...]
    hi_rep = jnp.broadcast_to(hi[:, None, :], (ROWS, K, C)).reshape(ROWS * K, C)
    dj = (xjs_ref[...] - hi_rep).astype(jnp.bfloat16)
    e = jnp.concatenate([hi_rep.astype(jnp.bfloat16), dj], axis=1)
    c = jax.lax.dot_general(e, ws_ref[...], (((1,), (0,)), ((), ())),
                            preferred_element_type=jnp.float32)
    hk = jnp.maximum(c + bs_ref[...], 0.0).reshape(ROWS, K, 4 * C)
    up = jnp.max(hk, axis=1).astype(jnp.bfloat16)
    for r in range(R):
        ur = up[:, C * r:C * (r + 1)]
        t = jnp.maximum(
            jax.lax.dot_general(ur, w1_ref[...], (((1,), (0,)), ((), ())),
                                preferred_element_type=jnp.float32)
            + b1_ref[...], 0.0).astype(jnp.bfloat16)
        o_ref[r] = jax.lax.dot_general(
            t, w2_ref[...], (((1,), (0,)), ((), ())),
            preferred_element_type=jnp.float32) + b2_ref[...]


_tail_cloud = pl.pallas_call(
    _tail_body,
    grid=(RB,),
    in_specs=[
        pl.BlockSpec((ROWS, C), lambda r: (r, 0)),
        pl.BlockSpec((ROWS * K, C), lambda r: (r, 0)),
        pl.BlockSpec((2 * C, 4 * C), lambda r: (0, 0)),
        pl.BlockSpec((1, 4 * C), lambda r: (0, 0)),
        pl.BlockSpec((C, C), lambda r: (0, 0)),
        pl.BlockSpec((1, C), lambda r: (0, 0)),
        pl.BlockSpec((C, 3), lambda r: (0, 0)),
        pl.BlockSpec((1, 3), lambda r: (0, 0)),
    ],
    out_specs=pl.BlockSpec((R, ROWS, 3), lambda r: (0, r, 0)),
    out_shape=jax.ShapeDtypeStruct((R, P, 3), jnp.float32),
)


def kernel(x, batch, We1, be1, We2, be2, Wf, bf, Ws, bs, Wr1, br1, Wr2, br2):
    be1r, be2r = be1.reshape(1, C), be2.reshape(1, C)
    bfr, bsr = bf.reshape(1, C), bs.reshape(1, 4 * C)
    br1r, br2r = br1.reshape(1, C), br2.reshape(1, 3)
    sqx = _rowsq(x).reshape(B, 1, P)
    knn2 = _knn_cloud_fn(32, True)
    knn1 = _knn_cloud_fn(17, False)
    outs = []
    for b in range(B):
        xb = x[b * P:(b + 1) * P]
        i1, i2 = knn2(xb, xb, sqx[b])
        xpad = jnp.pad(xb, ((0, PC - P), (0, 0)))
        XJ1, XJ2 = _gather2_sc(_pad_idx(i1), _pad_idx(i2), xpad)
        h, sqh = _conv12_cloud(xb, XJ1[:P * K], XJ2[:P * K],
                               We1, be1r, We2, be2r, Wf, bfr)
        (i_s,) = knn1(h, h, sqh.reshape(1, P))
        hpad = jnp.pad(h, ((0, PC - P), (0, 0)))
        XJS = _gather1_sc(_pad_idx(i_s), hpad)
        out4 = _tail_cloud(h, XJS[:P * K], Ws, bsr, Wr1, br1r, Wr2, br2r)
        outs.append(out4.transpose(1, 0, 2))
    return jnp.concatenate(outs, axis=0).reshape(N * R, 3)


# speedup vs baseline: 1.0264x; 1.0140x over previous
import functools

import jax
import jax.numpy as jnp
from jax import lax
from jax.experimental import pallas as pl
from jax.experimental.pallas import tpu as pltpu
from jax.experimental.pallas import tpu_sc as plsc

B, P, C, K, R = 5, 2000, 128, 16, 4
N = B * P
ROWS = 400
RB = P // ROWS

_SC = plsc.get_sparse_core_info()
_NC, _NS = _SC.num_cores, _SC.num_subcores
NW = _NC * _NS
PC = 2048
PPW = PC // NW
CH = 8
NCHUNK = PPW // CH


def _dot(a, b):
    return jax.lax.dot_general(a, b, (((1,), (0,)), ((), ())),
                               preferred_element_type=jnp.float32)


def _sq_body(x_ref, sq_ref):
    x = x_ref[...]
    sq_ref[...] = jnp.sum(x * x, axis=1, keepdims=True)


def _rowsq(x):
    return pl.pallas_call(
        _sq_body,
        grid=(N // ROWS,),
        in_specs=[pl.BlockSpec((ROWS, C), lambda r: (r, 0))],
        out_specs=pl.BlockSpec((ROWS, 1), lambda r: (r, 0)),
        out_shape=jax.ShapeDtypeStruct((N, 1), jnp.float32),
    )(x)


KR = 1000
KRB = P // KR


def _make_knn_body(T, want2):
    def body(xr_ref, xf_ref, sq_ref, *o_refs):
        xr = xr_ref[...]
        xf = xf_ref[...]
        sq = sq_ref[...]
        sqr = jnp.sum(xr * xr, axis=1, keepdims=True)
        d0 = sqr + sq - 2.0 * jax.lax.dot_general(
            xr, xf, (((1,), (1,)), ((), ())),
            preferred_element_type=jnp.float32)
        lane = jax.lax.broadcasted_iota(jnp.int32, (KR, P), 1)
        col16 = jax.lax.broadcasted_iota(jnp.int32, (KR, 16), 1)

        def step(t, carry):
            d, a1, a2 = carry
            m = jnp.min(d, axis=1, keepdims=True)
            sel = jnp.where(d == m, lane, jnp.int32(P))
            argi = jnp.min(sel, axis=1, keepdims=True)
            d = jnp.where(lane == argi, jnp.inf, d)
            a1 = jnp.where(col16 == (t - 1), argi, a1)
            if want2:
                upd2 = jnp.logical_and((t % 2) == 1, col16 == ((t - 1) // 2))
                a2 = jnp.where(upd2, argi, a2)
            return d, a1, a2

        z = jnp.zeros((KR, 16), jnp.int32)
        _, a1, a2 = lax.fori_loop(0, T, step, (d0, z, z))
        o_refs[0][...] = a1
        if want2:
            o_refs[1][...] = a2
    return body


@functools.cache
def _knn_cloud_fn(T, want2):
    n_out = 2 if want2 else 1
    return pl.pallas_call(
        _make_knn_body(T, want2),
        grid=(KRB,),
        in_specs=[
            pl.BlockSpec((KR, C), lambda r: (r, 0)),
            pl.BlockSpec((P, C), lambda r: (0, 0)),
            pl.BlockSpec((1, P), lambda r: (0, 0)),
        ],
        out_specs=[pl.BlockSpec((KR, 16), lambda r: (r, 0))
                   for _ in range(n_out)],
        out_shape=[jax.ShapeDtypeStruct((P, 16), jnp.int32)
                   for _ in range(n_out)],
    )


def _gather_loop(idx_hbm, tab_hbm, out_hbm, idx_v, rows_v, sem, base):
    def chunk_body(i, carry):
        rbase = (base + i * CH) * K
        pltpu.sync_copy(idx_hbm.at[pl.ds(rbase, CH * K)], idx_v)
        pltpu.async_copy(tab_hbm.at[idx_v], rows_v, sem).wait()
        pltpu.sync_copy(rows_v, out_hbm.at[pl.ds(rbase, CH * K)])
        return carry

    lax.fori_loop(0, NCHUNK, chunk_body, 0)


def _sc_mesh():
    return plsc.VectorSubcoreMesh(core_axis_name="c", subcore_axis_name="s")


@functools.partial(
    pl.kernel, mesh=_sc_mesh(),
    out_type=[jax.ShapeDtypeStruct((PC * K, C), jnp.float32),
              jax.ShapeDtypeStruct((PC * K, C), jnp.float32)],
    scratch_types=[
        pltpu.VMEM((CH * K,), jnp.int32),
        pltpu.VMEM((CH * K, C), jnp.float32),
        pltpu.SemaphoreType.DMA,
    ],
)
def _gather2_sc(idx1_hbm, idx2_hbm, tab_hbm, out1_hbm, out2_hbm,
                idx_v, rows_v, sem):
    wid = lax.axis_index("s") * _NC + lax.axis_index("c")
    base = wid * PPW
    _gather_loop(idx1_hbm, tab_hbm, out1_hbm, idx_v, rows_v, sem, base)
    _gather_loop(idx2_hbm, tab_hbm, out2_hbm, idx_v, rows_v, sem, base)


@functools.partial(
    pl.kernel, mesh=_sc_mesh(),
    out_type=jax.ShapeDtypeStruct((PC * K, C), jnp.float32),
    scratch_types=[
        pltpu.VMEM((CH * K,), jnp.int32),
        pltpu.VMEM((CH * K, C), jnp.float32),
        pltpu.SemaphoreType.DMA,
    ],
)
def _gather1_sc(idx_hbm, tab_hbm, out_hbm, idx_v, rows_v, sem):
    wid = lax.axis_index("s") * _NC + lax.axis_index("c")
    _gather_loop(idx_hbm, tab_hbm, out_hbm, idx_v, rows_v, sem, wid * PPW)


def _pad_idx(idx):
    return jnp.pad(idx, ((0, PC - P), (0, 0))).reshape(-1)


def _conv12_body(x_ref, xj1_ref, xj2_ref, we1_ref, be1_ref, we2_ref, be2_ref,
                 wf_ref, bf_ref, h_ref, sqh_ref):
    xi = x_ref[...]
    xi_rep = jnp.broadcast_to(xi[:, None, :], (ROWS, K, C)).reshape(ROWS * K, C)
    xi_b = xi_rep.astype(jnp.bfloat16)

    def conv(xj_ref, w_ref, b_ref):
        dj = (xj_ref[...] - xi_rep).astype(jnp.bfloat16)
        e = jnp.concatenate([xi_b, dj], axis=1)
        c = jax.lax.dot_general(e, w_ref[...], (((1,), (0,)), ((), ())),
                                preferred_element_type=jnp.float32)
        hk = jnp.maximum(c + b_ref[...], 0.0).reshape(ROWS, K, C)
        return jnp.max(hk, axis=1)

    h1 = conv(xj1_ref, we1_ref, be1_ref)
    h2 = conv(xj2_ref, we2_ref, be2_ref)
    s = h1 + h2
    h = jnp.maximum(_dot(s, wf_ref[...]) + bf_ref[...], 0.0) + xi
    h_ref[...] = h
    sqh_ref[...] = jnp.sum(h * h, axis=1, keepdims=True)


_conv12_cloud = pl.pallas_call(
    _conv12_body,
    grid=(RB,),
    in_specs=[
        pl.BlockSpec((ROWS, C), lambda r: (r, 0)),
        pl.BlockSpec((ROWS * K, C), lambda r: (r, 0)),
        pl.BlockSpec((ROWS * K, C), lambda r: (r, 0)),
        pl.BlockSpec((2 * C, C), lambda r: (0, 0)),
        pl.BlockSpec((1, C), lambda r: (0, 0)),
        pl.BlockSpec((2 * C, C), lambda r: (0, 0)),
        pl.BlockSpec((1, C), lambda r: (0, 0)),
        pl.BlockSpec((C, C), lambda r: (0, 0)),
        pl.BlockSpec((1, C), lambda r: (0, 0)),
    ],
    out_specs=[
        pl.BlockSpec((ROWS, C), lambda r: (r, 0)),
        pl.BlockSpec((ROWS, 1), lambda r: (r, 0)),
    ],
    out_shape=[
        jax.ShapeDtypeStruct((P, C), jnp.float32),
        jax.ShapeDtypeStruct((P, 1), jnp.float32),
    ],
)


def _tail_body(h_ref, xjs_ref, ws_ref, bs_ref, w1_ref, b1_ref, w2_ref, b2_ref,
               o_ref):
    hi = h_ref[...]
    hi_rep = jnp.broadcast_to(hi[:, None, :], (ROWS, K, C)).reshape(ROWS * K, C)
    dj = (xjs_ref[...] - hi_rep).astype(jnp.bfloat16)
    e = jnp.concatenate([hi_rep.astype(jnp.bfloat16), dj], axis=1)
    c = jax.lax.dot_general(e, ws_ref[...], (((1,), (0,)), ((), ())),
                            preferred_element_type=jnp.float32)
    hk = jnp.maximum(c + bs_ref[...], 0.0).reshape(ROWS, K, 4 * C)
    up = jnp.max(hk, axis=1).astype(jnp.bfloat16)
    for r in range(R):
        ur = up[:, C * r:C * (r + 1)]
        t = jnp.maximum(
            jax.lax.dot_general(ur, w1_ref[...], (((1,), (0,)), ((), ())),
                                preferred_element_type=jnp.float32)
            + b1_ref[...], 0.0).astype(jnp.bfloat16)
        o_ref[r] = jax.lax.dot_general(
            t, w2_ref[...], (((1,), (0,)), ((), ())),
            preferred_element_type=jnp.float32) + b2_ref[...]


_tail_cloud = pl.pallas_call(
    _tail_body,
    grid=(RB,),
    in_specs=[
        pl.BlockSpec((ROWS, C), lambda r: (r, 0)),
        pl.BlockSpec((ROWS * K, C), lambda r: (r, 0)),
        pl.BlockSpec((2 * C, 4 * C), lambda r: (0, 0)),
        pl.BlockSpec((1, 4 * C), lambda r: (0, 0)),
        pl.BlockSpec((C, C), lambda r: (0, 0)),
        pl.BlockSpec((1, C), lambda r: (0, 0)),
        pl.BlockSpec((C, 3), lambda r: (0, 0)),
        pl.BlockSpec((1, 3), lambda r: (0, 0)),
    ],
    out_specs=pl.BlockSpec((R, ROWS, 3), lambda r: (0, r, 0)),
    out_shape=jax.ShapeDtypeStruct((R, P, 3), jnp.float32),
)


def kernel(x, batch, We1, be1, We2, be2, Wf, bf, Ws, bs, Wr1, br1, Wr2, br2):
    be1r, be2r = be1.reshape(1, C), be2.reshape(1, C)
    bfr, bsr = bf.reshape(1, C), bs.reshape(1, 4 * C)
    br1r, br2r = br1.reshape(1, C), br2.reshape(1, 3)
    sqx = _rowsq(x).reshape(B, 1, P)
    knn2 = _knn_cloud_fn(32, True)
    knn1 = _knn_cloud_fn(17, False)
    outs = []
    for b in range(B):
        xb = x[b * P:(b + 1) * P]
        i1, i2 = knn2(xb, xb, sqx[b])
        xpad = jnp.pad(xb, ((0, PC - P), (0, 0)))
        XJ1, XJ2 = _gather2_sc(_pad_idx(i1), _pad_idx(i2), xpad)
        h, sqh = _conv12_cloud(xb, XJ1[:P * K], XJ2[:P * K],
                               We1, be1r, We2, be2r, Wf, bfr)
        (i_s,) = knn1(h, h, sqh.reshape(1, P))
        hpad = jnp.pad(h, ((0, PC - P), (0, 0)))
        XJS = _gather1_sc(_pad_idx(i_s), hpad)
        out4 = _tail_cloud(h, XJS[:P * K], Ws, bsr, Wr1, br1r, Wr2, br2r)
        outs.append(out4.transpose(1, 0, 2))
    return jnp.concatenate(outs, axis=0).reshape(N * R, 3)
